# Initial kernel scaffold; baseline (speedup 1.0000x reference)
import jax
import jax.numpy as jnp
from jax.experimental import pallas as pl


def kernel(x, edge_index, Wl1, bl1, Wr1, Wl2, bl2, Wr2, Wl3, bl3, Wr3, Wl4, bl4, Wr4, Wl5, bl5, Wr5):
    raise NotImplementedError("write your pallas kernel here")


# speedup vs baseline: 4.4395x; 4.4395x over previous
import functools

import jax
import jax.numpy as jnp
from jax import lax
from jax.experimental import pallas as pl
from jax.experimental.pallas import tpu as pltpu
from jax.experimental.pallas import tpu_sc as plsc

N = 10000
D = 128
E = 320000
NC = 2
NS = 16
NW = NC * NS
K = 128
CH = -(-E // (NW * K))
EPW = CH * K
EPAD = NW * EPW
ROWS_PT = 640
NPAD = ROWS_PT * NS
LAST = N - (NS - 1) * ROWS_PT

_MESH = plsc.VectorSubcoreMesh(core_axis_name="c", subcore_axis_name="s")


def _sc_agg_body(h_hbm, src_hbm, dst_hbm, agg_hbm, deg_hbm,
                 src_v, dst_v, rows_v, ones_v, zdeg_v, agg_s, deg_s, sem):
    c = lax.axis_index("c")
    s = lax.axis_index("s")
    wid = c * NS + s

    pltpu.sync_copy(src_hbm.at[wid], src_v)
    pltpu.sync_copy(dst_hbm.at[wid], dst_v)

    def _fill_row(r, _):
        for l in range(D // 16):
            rows_v[r, pl.ds(l * 16, 16)] = jnp.zeros((16,), jnp.float32)
        return 0
    lax.fori_loop(0, K, _fill_row, 0)

    def _fill_small(i, _):
        ones_v[pl.ds(i * 16, 16)] = jnp.full((16,), 1.0, jnp.float32)
        return 0
    lax.fori_loop(0, K // 16, _fill_small, 0)

    def _fill_zdeg(i, _):
        zdeg_v[pl.ds(i * 16, 16)] = jnp.zeros((16,), jnp.float32)
        return 0
    lax.fori_loop(0, ROWS_PT // 16, _fill_zdeg, 0)

    base = s * ROWS_PT
    for b in range(ROWS_PT // K):
        pltpu.sync_copy(rows_v, agg_s.at[pl.ds(base + b * K, K)])
    pltpu.sync_copy(zdeg_v, deg_s.at[pl.ds(base, ROWS_PT)])
    plsc.subcore_barrier()

    def _chunk(j, _):
        pltpu.async_copy(h_hbm.at[src_v.at[j]], rows_v, sem).wait()
        pltpu.sync_copy(rows_v, agg_s.at[dst_v.at[j]], add=True)
        pltpu.sync_copy(ones_v, deg_s.at[dst_v.at[j]], add=True)
        return 0
    lax.fori_loop(0, CH, _chunk, 0)
    plsc.subcore_barrier()

    pltpu.sync_copy(deg_s.at[pl.ds(base, ROWS_PT)], zdeg_v)

    @pl.when(s < NS - 1)
    def _():
        pltpu.sync_copy(agg_s.at[pl.ds(base, ROWS_PT)],
                        agg_hbm.at[c, pl.ds(base, ROWS_PT)])
        pltpu.sync_copy(zdeg_v, deg_hbm.at[pl.ds(c * N + base, ROWS_PT)])

    @pl.when(s == NS - 1)
    def _():
        pltpu.sync_copy(agg_s.at[pl.ds(base, LAST)],
                        agg_hbm.at[c, pl.ds(base, LAST)])
        pltpu.sync_copy(zdeg_v.at[pl.ds(0, LAST)],
                        deg_hbm.at[pl.ds(c * N + base, LAST)])


_sc_agg = functools.partial(
    pl.kernel,
    out_type=(jax.ShapeDtypeStruct((NC, N, D), jnp.float32),
              jax.ShapeDtypeStruct((NC * N,), jnp.float32)),
    mesh=_MESH,
    scratch_types=[
        pltpu.VMEM((CH, K), jnp.int32),
        pltpu.VMEM((CH, K), jnp.int32),
        pltpu.VMEM((K, D), jnp.float32),
        pltpu.VMEM((K,), jnp.float32),
        pltpu.VMEM((ROWS_PT,), jnp.float32),
        pltpu.VMEM_SHARED((NPAD, D), jnp.float32),
        pltpu.VMEM_SHARED((NPAD,), jnp.float32),
        pltpu.SemaphoreType.DMA,
    ],
)(_sc_agg_body)


R = 1000


def _tc_layer_body(relu, agg_ref, degt_ref, h_ref, wl_ref, bl_ref, wr_ref, o_ref):
    aggsum = agg_ref[0] + agg_ref[1]
    deg = degt_ref[:, 0] + degt_ref[:, 1]
    invd = 1.0 / jnp.maximum(deg, 1.0)
    m = aggsum * invd[:, None]
    out = lax.dot_general(m, wl_ref[...], (((1,), (1,)), ((), ())),
                          preferred_element_type=jnp.float32)
    out = out + bl_ref[...]
    out = out + lax.dot_general(h_ref[...], wr_ref[...], (((1,), (1,)), ((), ())),
                                preferred_element_type=jnp.float32)
    if relu:
        out = jnp.maximum(out, 0.0)
    o_ref[...] = out


def _tc_layer(relu):
    return pl.pallas_call(
        functools.partial(_tc_layer_body, relu),
        grid=(N // R,),
        in_specs=[
            pl.BlockSpec((NC, R, D), lambda i: (0, i, 0)),
            pl.BlockSpec((R, NC), lambda i: (i, 0)),
            pl.BlockSpec((R, D), lambda i: (i, 0)),
            pl.BlockSpec((D, D), lambda i: (0, 0)),
            pl.BlockSpec((1, D), lambda i: (0, 0)),
            pl.BlockSpec((D, D), lambda i: (0, 0)),
        ],
        out_specs=pl.BlockSpec((R, D), lambda i: (i, 0)),
        out_shape=jax.ShapeDtypeStruct((N, D), jnp.float32),
    )


def kernel(x, edge_index, Wl1, bl1, Wr1, Wl2, bl2, Wr2, Wl3, bl3, Wr3,
           Wl4, bl4, Wr4, Wl5, bl5, Wr5):
    src = edge_index[0]
    dst = edge_index[1]
    pad = EPAD - E
    srcp = jnp.concatenate([src, jnp.zeros((pad,), jnp.int32)]).reshape(NW, CH, K)
    dstp = jnp.concatenate([dst, jnp.full((pad,), N, jnp.int32)]).reshape(NW, CH, K)

    layers = [(Wl1, bl1, Wr1), (Wl2, bl2, Wr2), (Wl3, bl3, Wr3),
              (Wl4, bl4, Wr4), (Wl5, bl5, Wr5)]
    h = x
    for i, (Wl, bl, Wr) in enumerate(layers):
        agg, deg = _sc_agg(h, srcp, dstp)
        h = _tc_layer(i < 4)(agg, deg.reshape(NC, N).T, h, Wl,
                             bl.reshape(1, D), Wr)
    return h
